# baseline (speedup 1.0000x reference)
import functools
import jax
import jax.numpy as jnp
from jax import lax
from jax.experimental import pallas as pl
from jax.experimental.pallas import tpu as pltpu
from jax.experimental.pallas import tpu_sc as plsc

N_NODES = 10000
N_EDGES = 320000
D = 128

NC = 2
NS = 16
NW = NC * NS
EB = 128
EPW = N_EDGES // NW
NB = 80
EPW_PAD = NB * EB
ACC_ROWS = 10112
ROWS_PER_SUB = ACC_ROWS // NS
TRASH_ROW = N_NODES
DEG_ROWS = 10240
RPS = DEG_ROWS // NS

_mesh = plsc.VectorSubcoreMesh(core_axis_name="c", subcore_axis_name="s")


@functools.partial(
    pl.kernel,
    mesh=_mesh,
    compiler_params=pltpu.CompilerParams(needs_layout_passes=False),
    out_type=jax.ShapeDtypeStruct((NC, DEG_ROWS), jnp.float32),
    scratch_types=[
        pltpu.VMEM((NB, EB), jnp.int32),
        pltpu.VMEM((DEG_ROWS,), jnp.float32),
        pltpu.VMEM((NS, RPS), jnp.float32),
        pltpu.VMEM((RPS,), jnp.float32),
        pltpu.VMEM_SHARED((NS, NS, RPS), jnp.float32),
    ],
)
def _deg_kernel(tar_hbm, out_hbm, tar_v, hist, red, res, acc):
    c = lax.axis_index("c")
    s = lax.axis_index("s")
    w = c * NS + s
    pltpu.sync_copy(tar_hbm.at[w], tar_v)
    zero16 = jnp.zeros((16,), jnp.float32)

    def zbody(k, carry):
        hist[pl.ds(k * 16, 16)] = zero16
        return carry

    lax.fori_loop(0, DEG_ROWS // 16, zbody, 0)
    ones16 = jnp.ones((16,), jnp.float32)

    def sbody(j, carry):
        for k in range(EB // 16):
            iv = tar_v[j, pl.ds(k * 16, 16)]
            plsc.addupdate_scatter(hist, [iv], ones16)
        return carry

    lax.fori_loop(0, NB, sbody, 0)
    for t in range(NS):
        pltpu.sync_copy(hist.at[pl.ds(t * RPS, RPS)], acc.at[t, s])
    plsc.subcore_barrier()
    pltpu.sync_copy(acc.at[s], red)

    def rbody(k, carry):
        v = red[0, pl.ds(k * 16, 16)]
        for i in range(1, NS):
            v = v + red[i, pl.ds(k * 16, 16)]
        res[pl.ds(k * 16, 16)] = v
        return carry

    lax.fori_loop(0, RPS // 16, rbody, 0)
    pltpu.sync_copy(res, out_hbm.at[c, pl.ds(s * RPS, RPS)])


CHUNK = 16
NCHUNK = NB // CHUNK


@functools.partial(
    pl.kernel,
    mesh=_mesh,
    compiler_params=pltpu.CompilerParams(use_tc_tiling_on_sc=False,
                                         needs_layout_passes=False),
    out_type=jax.ShapeDtypeStruct((NC, ACC_ROWS, D), jnp.float32),
    scratch_types=[
        pltpu.VMEM((CHUNK, EB), jnp.int32),
        pltpu.VMEM((2 * CHUNK, EB // 2), jnp.int32),
        pltpu.VMEM((EB, D // 2), jnp.int32),
        pltpu.VMEM((EB, D // 2), jnp.int32),
        pltpu.VMEM((EB // 2, D), jnp.float32),
        pltpu.VMEM((EB // 2, D), jnp.float32),
        pltpu.SemaphoreType.DMA,
        pltpu.SemaphoreType.DMA,
        pltpu.SemaphoreType.DMA,
        pltpu.SemaphoreType.DMA,
        pltpu.VMEM_SHARED((ACC_ROWS, D), jnp.float32),
    ],
)
def _agg_kernel(gpk_hbm, src_hbm, tar_hbm, zeros_hbm, out_hbm,
                src_v, tar_v, pbuf0, pbuf1, fb0, fb1, g0, g1, s0, s1, acc):
    c = lax.axis_index("c")
    s = lax.axis_index("s")
    w = c * NS + s
    for k in range(4):
        pltpu.sync_copy(zeros_hbm, acc.at[pl.ds(s * ROWS_PER_SUB + k * EB, EB)])
    pltpu.sync_copy(zeros_hbm.at[pl.ds(0, ROWS_PER_SUB - 4 * EB)],
                    acc.at[pl.ds(s * ROWS_PER_SUB + 4 * EB, ROWS_PER_SUB - 4 * EB)])
    plsc.subcore_barrier()

    def gstart(jv, buf, sem):
        pltpu.async_copy(gpk_hbm.at[src_v.at[jv]], buf, sem)

    def sstart(jh, fb, sem):
        pltpu.async_copy(fb, acc.at[tar_v.at[jh]], sem, add=True)

    def gdrain(buf, sem):
        pltpu.make_async_copy(gpk_hbm.at[pl.ds(0, EB)], buf, sem).wait()

    def sdrain(fb, sem):
        pltpu.make_async_copy(gpk_hbm.at[pl.ds(0, EB // 2)], fb, sem).wait()

    HB = EB // 2

    def unpack_half(buf, h, fb):
        def urow(r4, carry):
            for dr in range(4):
                for k in range(D // 32):
                    v = buf[h * HB + r4 * 4 + dr, pl.ds(k * 16, 16)]
                    fb[r4 * 4 + dr, pl.ds(k * 16, 16)] = plsc.bitcast(
                        lax.shift_left(v, 16), jnp.float32)
                    fb[r4 * 4 + dr, pl.ds(64 + k * 16, 16)] = plsc.bitcast(
                        lax.bitwise_and(v, jnp.int32(-65536)), jnp.float32)
            return carry

        lax.fori_loop(0, HB // 4, urow, 0)

    HALF = CHUNK // 2
    for ci in range(NCHUNK):
        pltpu.sync_copy(src_hbm.at[w, pl.ds(ci * CHUNK, CHUNK)], src_v)
        pltpu.sync_copy(tar_hbm.at[w, pl.ds(ci * 2 * CHUNK, 2 * CHUNK)], tar_v)
        gstart(0, pbuf0, g0)
        gstart(1, pbuf1, g1)

        def pair(j, carry):
            gdrain(pbuf0, g0)
            unpack_half(pbuf0, 0, fb0)
            sstart(4 * j + 0, fb0, s0)
            unpack_half(pbuf0, 1, fb1)
            sstart(4 * j + 1, fb1, s1)

            @pl.when(j < HALF - 1)
            def _():
                gstart(2 * j + 2, pbuf0, g0)

            gdrain(pbuf1, g1)
            sdrain(fb0, s0)
            unpack_half(pbuf1, 0, fb0)
            sstart(4 * j + 2, fb0, s0)
            sdrain(fb1, s1)
            unpack_half(pbuf1, 1, fb1)
            sstart(4 * j + 3, fb1, s1)

            @pl.when(j < HALF - 1)
            def _():
                gstart(2 * j + 3, pbuf1, g1)

            sdrain(fb0, s0)
            sdrain(fb1, s1)
            return carry

        lax.fori_loop(0, HALF, pair, 0)
    plsc.subcore_barrier()
    for k in range(4):
        r = s * ROWS_PER_SUB + k * EB
        pltpu.sync_copy(acc.at[pl.ds(r, EB)], out_hbm.at[c, pl.ds(r, EB)])
    r = s * ROWS_PER_SUB + 4 * EB
    pltpu.sync_copy(acc.at[pl.ds(r, ROWS_PER_SUB - 4 * EB)],
                    out_hbm.at[c, pl.ds(r, ROWS_PER_SUB - 4 * EB)])


def _proj_body(x_ref, w_ref, b_ref, deg_ref, g_ref, gpk_ref):
    h = lax.dot_general(x_ref[...], w_ref[...],
                        (((1,), (1,)), ((), ())),
                        preferred_element_type=jnp.float32)
    h = h + b_ref[...]
    degsum = deg_ref[0] + deg_ref[1] + 1.0
    g = lax.rsqrt(degsum) * h
    g_ref[...] = g
    gb = g.astype(jnp.bfloat16)
    lo = lax.convert_element_type(
        lax.bitcast_convert_type(gb[:, : D // 2], jnp.uint16), jnp.uint32)
    hi = lax.convert_element_type(
        lax.bitcast_convert_type(gb[:, D // 2:], jnp.uint16), jnp.uint32)
    packed = lax.bitwise_or(lo, lax.shift_left(hi, jnp.uint32(16)))
    gpk_ref[...] = lax.bitcast_convert_type(packed, jnp.int32)


def _final_body(p_ref, g_ref, deg_ref, o_ref):
    degsum = deg_ref[0] + deg_ref[1] + 1.0
    dis = lax.rsqrt(degsum)
    o_ref[...] = dis * (p_ref[0] + p_ref[1] + g_ref[...])


def kernel(x, edge_index, W, b):
    src = edge_index[1].astype(jnp.int32)
    tar = edge_index[0].astype(jnp.int32)
    pad = EPW_PAD - EPW
    src_p = jnp.pad(src.reshape(NW, EPW), ((0, 0), (0, pad))).reshape(NW, NB, EB)
    tar_p = jnp.pad(tar.reshape(NW, EPW), ((0, 0), (0, pad)),
                    constant_values=TRASH_ROW).reshape(NW, NB, EB)
    zerosD = jnp.zeros((EB, D), jnp.float32)

    deg_p = _deg_kernel(tar_p).reshape(NC, DEG_ROWS, 1)

    RB = 1024
    grid = (10,)
    g, gpk = pl.pallas_call(
        _proj_body,
        grid=grid,
        in_specs=[
            pl.BlockSpec((RB, D), lambda i: (i, 0)),
            pl.BlockSpec((D, D), lambda i: (0, 0)),
            pl.BlockSpec((1, D), lambda i: (0, 0)),
            pl.BlockSpec((NC, RB, 1), lambda i: (0, i, 0)),
        ],
        out_specs=[
            pl.BlockSpec((RB, D), lambda i: (i, 0)),
            pl.BlockSpec((RB, D // 2), lambda i: (i, 0)),
        ],
        out_shape=[
            jax.ShapeDtypeStruct((N_NODES, D), jnp.float32),
            jax.ShapeDtypeStruct((N_NODES, D // 2), jnp.int32),
        ],
    )(x, W, b.reshape(1, D), deg_p)

    partials = _agg_kernel(gpk, src_p, tar_p.reshape(NW, 2 * NB, EB // 2), zerosD)

    out = pl.pallas_call(
        _final_body,
        grid=grid,
        in_specs=[
            pl.BlockSpec((NC, RB, D), lambda i: (0, i, 0)),
            pl.BlockSpec((RB, D), lambda i: (i, 0)),
            pl.BlockSpec((NC, RB, 1), lambda i: (0, i, 0)),
        ],
        out_specs=pl.BlockSpec((RB, D), lambda i: (i, 0)),
        out_shape=jax.ShapeDtypeStruct((N_NODES, D), jnp.float32),
    )(partials, g, deg_p)
    return out


# speedup vs baseline: 1.0266x; 1.0266x over previous
import functools
import jax
import jax.numpy as jnp
from jax import lax
from jax.experimental import pallas as pl
from jax.experimental.pallas import tpu as pltpu
from jax.experimental.pallas import tpu_sc as plsc

N_NODES = 10000
N_EDGES = 320000
D = 128

NC = 2
NS = 16
NW = NC * NS
EB = 128
EPW = N_EDGES // NW
NB = 80
EPW_PAD = NB * EB
ACC_ROWS = 10112
ROWS_PER_SUB = ACC_ROWS // NS
TRASH_ROW = N_NODES
DEG_ROWS = 10240
RPS = DEG_ROWS // NS

_mesh = plsc.VectorSubcoreMesh(core_axis_name="c", subcore_axis_name="s")


@functools.partial(
    pl.kernel,
    mesh=_mesh,
    compiler_params=pltpu.CompilerParams(needs_layout_passes=False),
    out_type=jax.ShapeDtypeStruct((NC, DEG_ROWS), jnp.float32),
    scratch_types=[
        pltpu.VMEM((NB, EB), jnp.int32),
        pltpu.VMEM((DEG_ROWS,), jnp.float32),
        pltpu.VMEM((NS, RPS), jnp.float32),
        pltpu.VMEM((RPS,), jnp.float32),
        pltpu.VMEM_SHARED((NS, NS, RPS), jnp.float32),
    ],
)
def _deg_kernel(tar_hbm, out_hbm, tar_v, hist, red, res, acc):
    c = lax.axis_index("c")
    s = lax.axis_index("s")
    w = c * NS + s
    pltpu.sync_copy(tar_hbm.at[w], tar_v)
    zero16 = jnp.zeros((16,), jnp.float32)

    def zbody(k, carry):
        hist[pl.ds(k * 16, 16)] = zero16
        return carry

    lax.fori_loop(0, DEG_ROWS // 16, zbody, 0)
    ones16 = jnp.ones((16,), jnp.float32)

    def sbody(j, carry):
        for k in range(EB // 16):
            iv = tar_v[j, pl.ds(k * 16, 16)]
            plsc.addupdate_scatter(hist, [iv], ones16)
        return carry

    lax.fori_loop(0, NB, sbody, 0)
    for t in range(NS):
        pltpu.sync_copy(hist.at[pl.ds(t * RPS, RPS)], acc.at[t, s])
    plsc.subcore_barrier()
    pltpu.sync_copy(acc.at[s], red)

    def rbody(k, carry):
        v = red[0, pl.ds(k * 16, 16)]
        for i in range(1, NS):
            v = v + red[i, pl.ds(k * 16, 16)]
        res[pl.ds(k * 16, 16)] = v
        return carry

    lax.fori_loop(0, RPS // 16, rbody, 0)
    pltpu.sync_copy(res, out_hbm.at[c, pl.ds(s * RPS, RPS)])


CHUNK = 16
NCHUNK = NB // CHUNK


@functools.partial(
    pl.kernel,
    mesh=_mesh,
    compiler_params=pltpu.CompilerParams(use_tc_tiling_on_sc=False,
                                         needs_layout_passes=False),
    out_type=jax.ShapeDtypeStruct((NC, ACC_ROWS, D), jnp.float32),
    scratch_types=[
        pltpu.VMEM((CHUNK, EB), jnp.int32),
        pltpu.VMEM((2 * CHUNK, EB // 2), jnp.int32),
        pltpu.VMEM((EB, D // 2), jnp.int32),
        pltpu.VMEM((EB, D // 2), jnp.int32),
        pltpu.VMEM((EB // 2, D), jnp.float32),
        pltpu.VMEM((EB // 2, D), jnp.float32),
        pltpu.SemaphoreType.DMA,
        pltpu.SemaphoreType.DMA,
        pltpu.SemaphoreType.DMA,
        pltpu.SemaphoreType.DMA,
        pltpu.VMEM_SHARED((ACC_ROWS, D), jnp.float32),
    ],
)
def _agg_kernel(gpk_hbm, src_hbm, tar_hbm, zeros_hbm, out_hbm,
                src_v, tar_v, pbuf0, pbuf1, fb0, fb1, g0, g1, s0, s1, acc):
    c = lax.axis_index("c")
    s = lax.axis_index("s")
    w = c * NS + s
    for k in range(4):
        pltpu.sync_copy(zeros_hbm, acc.at[pl.ds(s * ROWS_PER_SUB + k * EB, EB)])
    pltpu.sync_copy(zeros_hbm.at[pl.ds(0, ROWS_PER_SUB - 4 * EB)],
                    acc.at[pl.ds(s * ROWS_PER_SUB + 4 * EB, ROWS_PER_SUB - 4 * EB)])
    plsc.subcore_barrier()

    def gstart(jv, buf, sem):
        pltpu.async_copy(gpk_hbm.at[src_v.at[jv]], buf, sem)

    def sstart(jh, fb, sem):
        pltpu.async_copy(fb, acc.at[tar_v.at[jh]], sem, add=True)

    def gdrain(buf, sem):
        pltpu.make_async_copy(gpk_hbm.at[pl.ds(0, EB)], buf, sem).wait()

    def sdrain(fb, sem):
        pltpu.make_async_copy(gpk_hbm.at[pl.ds(0, EB // 2)], fb, sem).wait()

    HB = EB // 2

    def unpack_half(buf, h, fb):
        def urow(r4, carry):
            for dr in range(4):
                for k in range(D // 32):
                    v = buf[h * HB + r4 * 4 + dr, pl.ds(k * 16, 16)]
                    fb[r4 * 4 + dr, pl.ds(k * 16, 16)] = plsc.bitcast(
                        lax.shift_left(v, 16), jnp.float32)
                    fb[r4 * 4 + dr, pl.ds(64 + k * 16, 16)] = plsc.bitcast(
                        lax.bitwise_and(v, jnp.int32(-65536)), jnp.float32)
            return carry

        lax.fori_loop(0, HB // 4, urow, 0)

    HALF = CHUNK // 2
    for ci in range(NCHUNK):
        pltpu.sync_copy(src_hbm.at[w, pl.ds(ci * CHUNK, CHUNK)], src_v)
        pltpu.sync_copy(tar_hbm.at[w, pl.ds(ci * 2 * CHUNK, 2 * CHUNK)], tar_v)
        gstart(0, pbuf0, g0)
        gstart(1, pbuf1, g1)

        def pair(j, carry):
            gdrain(pbuf0, g0)

            @pl.when(j > 0)
            def _():
                sdrain(fb0, s0)

            unpack_half(pbuf0, 0, fb0)
            sstart(4 * j + 0, fb0, s0)

            @pl.when(j > 0)
            def _():
                sdrain(fb1, s1)

            unpack_half(pbuf0, 1, fb1)
            sstart(4 * j + 1, fb1, s1)

            @pl.when(j < HALF - 1)
            def _():
                gstart(2 * j + 2, pbuf0, g0)

            gdrain(pbuf1, g1)
            sdrain(fb0, s0)
            unpack_half(pbuf1, 0, fb0)
            sstart(4 * j + 2, fb0, s0)
            sdrain(fb1, s1)
            unpack_half(pbuf1, 1, fb1)
            sstart(4 * j + 3, fb1, s1)

            @pl.when(j < HALF - 1)
            def _():
                gstart(2 * j + 3, pbuf1, g1)

            return carry

        lax.fori_loop(0, HALF, pair, 0)
        sdrain(fb0, s0)
        sdrain(fb1, s1)
    plsc.subcore_barrier()
    for k in range(4):
        r = s * ROWS_PER_SUB + k * EB
        pltpu.sync_copy(acc.at[pl.ds(r, EB)], out_hbm.at[c, pl.ds(r, EB)])
    r = s * ROWS_PER_SUB + 4 * EB
    pltpu.sync_copy(acc.at[pl.ds(r, ROWS_PER_SUB - 4 * EB)],
                    out_hbm.at[c, pl.ds(r, ROWS_PER_SUB - 4 * EB)])


def _proj_body(x_ref, w_ref, b_ref, deg_ref, g_ref, gpk_ref):
    h = lax.dot_general(x_ref[...], w_ref[...],
                        (((1,), (1,)), ((), ())),
                        preferred_element_type=jnp.float32)
    h = h + b_ref[...]
    degsum = deg_ref[0] + deg_ref[1] + 1.0
    g = lax.rsqrt(degsum) * h
    g_ref[...] = g
    gb = g.astype(jnp.bfloat16)
    lo = lax.convert_element_type(
        lax.bitcast_convert_type(gb[:, : D // 2], jnp.uint16), jnp.uint32)
    hi = lax.convert_element_type(
        lax.bitcast_convert_type(gb[:, D // 2:], jnp.uint16), jnp.uint32)
    packed = lax.bitwise_or(lo, lax.shift_left(hi, jnp.uint32(16)))
    gpk_ref[...] = lax.bitcast_convert_type(packed, jnp.int32)


def _final_body(p_ref, g_ref, deg_ref, o_ref):
    degsum = deg_ref[0] + deg_ref[1] + 1.0
    dis = lax.rsqrt(degsum)
    o_ref[...] = dis * (p_ref[0] + p_ref[1] + g_ref[...])


def kernel(x, edge_index, W, b):
    src = edge_index[1].astype(jnp.int32)
    tar = edge_index[0].astype(jnp.int32)
    pad = EPW_PAD - EPW
    src_p = jnp.pad(src.reshape(NW, EPW), ((0, 0), (0, pad))).reshape(NW, NB, EB)
    tar_p = jnp.pad(tar.reshape(NW, EPW), ((0, 0), (0, pad)),
                    constant_values=TRASH_ROW).reshape(NW, NB, EB)
    zerosD = jnp.zeros((EB, D), jnp.float32)

    deg_p = _deg_kernel(tar_p).reshape(NC, DEG_ROWS, 1)

    RB = 1024
    grid = (10,)
    g, gpk = pl.pallas_call(
        _proj_body,
        grid=grid,
        in_specs=[
            pl.BlockSpec((RB, D), lambda i: (i, 0)),
            pl.BlockSpec((D, D), lambda i: (0, 0)),
            pl.BlockSpec((1, D), lambda i: (0, 0)),
            pl.BlockSpec((NC, RB, 1), lambda i: (0, i, 0)),
        ],
        out_specs=[
            pl.BlockSpec((RB, D), lambda i: (i, 0)),
            pl.BlockSpec((RB, D // 2), lambda i: (i, 0)),
        ],
        out_shape=[
            jax.ShapeDtypeStruct((N_NODES, D), jnp.float32),
            jax.ShapeDtypeStruct((N_NODES, D // 2), jnp.int32),
        ],
    )(x, W, b.reshape(1, D), deg_p)

    partials = _agg_kernel(gpk, src_p, tar_p.reshape(NW, 2 * NB, EB // 2), zerosD)

    out = pl.pallas_call(
        _final_body,
        grid=grid,
        in_specs=[
            pl.BlockSpec((NC, RB, D), lambda i: (0, i, 0)),
            pl.BlockSpec((RB, D), lambda i: (i, 0)),
            pl.BlockSpec((NC, RB, 1), lambda i: (0, i, 0)),
        ],
        out_specs=pl.BlockSpec((RB, D), lambda i: (i, 0)),
        out_shape=jax.ShapeDtypeStruct((N_NODES, D), jnp.float32),
    )(partials, g, deg_p)
    return out


# speedup vs baseline: 1.0381x; 1.0111x over previous
import functools
import jax
import jax.numpy as jnp
from jax import lax
from jax.experimental import pallas as pl
from jax.experimental.pallas import tpu as pltpu
from jax.experimental.pallas import tpu_sc as plsc

N_NODES = 10000
N_EDGES = 320000
D = 128

NC = 2
NS = 16
NW = NC * NS
EB = 128
EPW = N_EDGES // NW
NB = 80
EPW_PAD = NB * EB
ACC_ROWS = 10112
ROWS_PER_SUB = ACC_ROWS // NS
TRASH_ROW = N_NODES
DEG_ROWS = 10240
RPS = DEG_ROWS // NS

_mesh = plsc.VectorSubcoreMesh(core_axis_name="c", subcore_axis_name="s")


@functools.partial(
    pl.kernel,
    mesh=_mesh,
    compiler_params=pltpu.CompilerParams(needs_layout_passes=False),
    out_type=jax.ShapeDtypeStruct((NC, DEG_ROWS), jnp.float32),
    scratch_types=[
        pltpu.VMEM((NB, EB), jnp.int32),
        pltpu.VMEM((DEG_ROWS,), jnp.float32),
        pltpu.VMEM((NS, RPS), jnp.float32),
        pltpu.VMEM((RPS,), jnp.float32),
        pltpu.VMEM_SHARED((NS, NS, RPS), jnp.float32),
    ],
)
def _deg_kernel(tar_hbm, out_hbm, tar_v, hist, red, res, acc):
    c = lax.axis_index("c")
    s = lax.axis_index("s")
    w = c * NS + s
    pltpu.sync_copy(tar_hbm.at[w], tar_v)
    zero16 = jnp.zeros((16,), jnp.float32)

    def zbody(k, carry):
        hist[pl.ds(k * 16, 16)] = zero16
        return carry

    lax.fori_loop(0, DEG_ROWS // 16, zbody, 0)
    ones16 = jnp.ones((16,), jnp.float32)

    def sbody(j, carry):
        for k in range(EB // 16):
            iv = tar_v[j, pl.ds(k * 16, 16)]
            plsc.addupdate_scatter(hist, [iv], ones16)
        return carry

    lax.fori_loop(0, NB, sbody, 0)
    for t in range(NS):
        pltpu.sync_copy(hist.at[pl.ds(t * RPS, RPS)], acc.at[t, s])
    plsc.subcore_barrier()
    pltpu.sync_copy(acc.at[s], red)

    def rbody(k, carry):
        v = red[0, pl.ds(k * 16, 16)]
        for i in range(1, NS):
            v = v + red[i, pl.ds(k * 16, 16)]
        res[pl.ds(k * 16, 16)] = v
        return carry

    lax.fori_loop(0, RPS // 16, rbody, 0)
    pltpu.sync_copy(res, out_hbm.at[c, pl.ds(s * RPS, RPS)])


CHUNK = 16
NCHUNK = NB // CHUNK


@functools.partial(
    pl.kernel,
    mesh=_mesh,
    compiler_params=pltpu.CompilerParams(use_tc_tiling_on_sc=False,
                                         needs_layout_passes=False),
    out_type=jax.ShapeDtypeStruct((NC, ACC_ROWS, D), jnp.float32),
    scratch_types=[
        pltpu.VMEM((CHUNK, EB), jnp.int32),
        pltpu.VMEM((2 * CHUNK, EB // 2), jnp.int32),
        pltpu.VMEM((CHUNK, EB), jnp.int32),
        pltpu.VMEM((2 * CHUNK, EB // 2), jnp.int32),
        pltpu.VMEM((EB, D // 2), jnp.int32),
        pltpu.VMEM((EB, D // 2), jnp.int32),
        pltpu.VMEM((EB // 2, D), jnp.float32),
        pltpu.VMEM((EB // 2, D), jnp.float32),
        pltpu.SemaphoreType.DMA,
        pltpu.SemaphoreType.DMA,
        pltpu.SemaphoreType.DMA,
        pltpu.SemaphoreType.DMA,
        pltpu.VMEM_SHARED((ACC_ROWS, D), jnp.float32),
    ],
)
def _agg_kernel(gpk_hbm, src_hbm, tar_hbm, zeros_hbm, out_hbm,
                src_v0, tar_v0, src_v1, tar_v1,
                pbuf0, pbuf1, fb0, fb1, g0, g1, s0, s1, acc):
    c = lax.axis_index("c")
    s = lax.axis_index("s")
    w = c * NS + s
    for k in range(4):
        pltpu.sync_copy(zeros_hbm, acc.at[pl.ds(s * ROWS_PER_SUB + k * EB, EB)])
    pltpu.sync_copy(zeros_hbm.at[pl.ds(0, ROWS_PER_SUB - 4 * EB)],
                    acc.at[pl.ds(s * ROWS_PER_SUB + 4 * EB, ROWS_PER_SUB - 4 * EB)])
    plsc.subcore_barrier()

    def gstart(src_v, jv, buf, sem):
        pltpu.async_copy(gpk_hbm.at[src_v.at[jv]], buf, sem)

    def sstart(tar_v, jh, fb, sem):
        pltpu.async_copy(fb, acc.at[tar_v.at[jh]], sem, add=True)

    def gdrain(buf, sem):
        pltpu.make_async_copy(gpk_hbm.at[pl.ds(0, EB)], buf, sem).wait()

    def sdrain(fb, sem):
        pltpu.make_async_copy(gpk_hbm.at[pl.ds(0, EB // 2)], fb, sem).wait()

    HB = EB // 2

    def unpack_half(buf, h, fb):
        def urow(r4, carry):
            for dr in range(4):
                for k in range(D // 32):
                    v = buf[h * HB + r4 * 4 + dr, pl.ds(k * 16, 16)]
                    fb[r4 * 4 + dr, pl.ds(k * 16, 16)] = plsc.bitcast(
                        lax.shift_left(v, 16), jnp.float32)
                    fb[r4 * 4 + dr, pl.ds(64 + k * 16, 16)] = plsc.bitcast(
                        lax.bitwise_and(v, jnp.int32(-65536)), jnp.float32)
            return carry

        lax.fori_loop(0, HB // 4, urow, 0)

    HALF = CHUNK // 2
    idxbufs = [(src_v0, tar_v0), (src_v1, tar_v1)]
    pltpu.sync_copy(src_hbm.at[w, pl.ds(0, CHUNK)], src_v0)
    pltpu.sync_copy(tar_hbm.at[w, pl.ds(0, 2 * CHUNK)], tar_v0)
    gstart(src_v0, 0, pbuf0, g0)
    gstart(src_v0, 1, pbuf1, g1)
    for ci in range(NCHUNK):
        src_v, tar_v = idxbufs[ci % 2]
        src_n, tar_n = idxbufs[(ci + 1) % 2]
        last = ci == NCHUNK - 1
        if not last:
            pltpu.sync_copy(src_hbm.at[w, pl.ds((ci + 1) * CHUNK, CHUNK)], src_n)
            pltpu.sync_copy(tar_hbm.at[w, pl.ds((ci + 1) * 2 * CHUNK, 2 * CHUNK)],
                            tar_n)

        def pair(j, carry):
            gdrain(pbuf0, g0)

            @pl.when(j > 0)
            def _():
                sdrain(fb0, s0)

            unpack_half(pbuf0, 0, fb0)
            sstart(tar_v, 4 * j + 0, fb0, s0)

            @pl.when(j > 0)
            def _():
                sdrain(fb1, s1)

            unpack_half(pbuf0, 1, fb1)
            sstart(tar_v, 4 * j + 1, fb1, s1)

            @pl.when(j < HALF - 1)
            def _():
                gstart(src_v, 2 * j + 2, pbuf0, g0)

            if not last:
                @pl.when(j == HALF - 1)
                def _():
                    gstart(src_n, 0, pbuf0, g0)

            gdrain(pbuf1, g1)
            sdrain(fb0, s0)
            unpack_half(pbuf1, 0, fb0)
            sstart(tar_v, 4 * j + 2, fb0, s0)
            sdrain(fb1, s1)
            unpack_half(pbuf1, 1, fb1)
            sstart(tar_v, 4 * j + 3, fb1, s1)

            @pl.when(j < HALF - 1)
            def _():
                gstart(src_v, 2 * j + 3, pbuf1, g1)

            if not last:
                @pl.when(j == HALF - 1)
                def _():
                    gstart(src_n, 1, pbuf1, g1)

            return carry

        lax.fori_loop(0, HALF, pair, 0)
        sdrain(fb0, s0)
        sdrain(fb1, s1)
    plsc.subcore_barrier()
    for k in range(4):
        r = s * ROWS_PER_SUB + k * EB
        pltpu.sync_copy(acc.at[pl.ds(r, EB)], out_hbm.at[c, pl.ds(r, EB)])
    r = s * ROWS_PER_SUB + 4 * EB
    pltpu.sync_copy(acc.at[pl.ds(r, ROWS_PER_SUB - 4 * EB)],
                    out_hbm.at[c, pl.ds(r, ROWS_PER_SUB - 4 * EB)])


def _proj_body(x_ref, w_ref, b_ref, deg_ref, g_ref, gpk_ref):
    h = lax.dot_general(x_ref[...], w_ref[...],
                        (((1,), (1,)), ((), ())),
                        preferred_element_type=jnp.float32)
    h = h + b_ref[...]
    degsum = deg_ref[0] + deg_ref[1] + 1.0
    g = lax.rsqrt(degsum) * h
    g_ref[...] = g
    gb = g.astype(jnp.bfloat16)
    lo = lax.convert_element_type(
        lax.bitcast_convert_type(gb[:, : D // 2], jnp.uint16), jnp.uint32)
    hi = lax.convert_element_type(
        lax.bitcast_convert_type(gb[:, D // 2:], jnp.uint16), jnp.uint32)
    packed = lax.bitwise_or(lo, lax.shift_left(hi, jnp.uint32(16)))
    gpk_ref[...] = lax.bitcast_convert_type(packed, jnp.int32)


def _final_body(p_ref, g_ref, deg_ref, o_ref):
    degsum = deg_ref[0] + deg_ref[1] + 1.0
    dis = lax.rsqrt(degsum)
    o_ref[...] = dis * (p_ref[0] + p_ref[1] + g_ref[...])


def kernel(x, edge_index, W, b):
    src = edge_index[1].astype(jnp.int32)
    tar = edge_index[0].astype(jnp.int32)
    pad = EPW_PAD - EPW
    src_p = jnp.pad(src.reshape(NW, EPW), ((0, 0), (0, pad))).reshape(NW, NB, EB)
    tar_p = jnp.pad(tar.reshape(NW, EPW), ((0, 0), (0, pad)),
                    constant_values=TRASH_ROW).reshape(NW, NB, EB)
    zerosD = jnp.zeros((EB, D), jnp.float32)

    deg_p = _deg_kernel(tar_p).reshape(NC, DEG_ROWS, 1)

    RB = 1024
    grid = (10,)
    g, gpk = pl.pallas_call(
        _proj_body,
        grid=grid,
        in_specs=[
            pl.BlockSpec((RB, D), lambda i: (i, 0)),
            pl.BlockSpec((D, D), lambda i: (0, 0)),
            pl.BlockSpec((1, D), lambda i: (0, 0)),
            pl.BlockSpec((NC, RB, 1), lambda i: (0, i, 0)),
        ],
        out_specs=[
            pl.BlockSpec((RB, D), lambda i: (i, 0)),
            pl.BlockSpec((RB, D // 2), lambda i: (i, 0)),
        ],
        out_shape=[
            jax.ShapeDtypeStruct((N_NODES, D), jnp.float32),
            jax.ShapeDtypeStruct((N_NODES, D // 2), jnp.int32),
        ],
    )(x, W, b.reshape(1, D), deg_p)

    partials = _agg_kernel(gpk, src_p, tar_p.reshape(NW, 2 * NB, EB // 2), zerosD)

    out = pl.pallas_call(
        _final_body,
        grid=grid,
        in_specs=[
            pl.BlockSpec((NC, RB, D), lambda i: (0, i, 0)),
            pl.BlockSpec((RB, D), lambda i: (i, 0)),
            pl.BlockSpec((NC, RB, 1), lambda i: (0, i, 0)),
        ],
        out_specs=pl.BlockSpec((RB, D), lambda i: (i, 0)),
        out_shape=jax.ShapeDtypeStruct((N_NODES, D), jnp.float32),
    )(partials, g, deg_p)
    return out
